# baseline (speedup 1.0000x reference)
import functools
import jax
import jax.numpy as jnp
from jax import lax
from jax.experimental import pallas as pl
from jax.experimental.pallas import tpu as pltpu
from jax.experimental.pallas import tpu_sc as plsc

HID = 1024
EPS = 1e-12

NC = 2
NS = 16
NW = NC * NS


def _sc_gather(word_table, ids_flat, n_rows, chunk):
    b_per_w = n_rows // NW
    n_chunks = b_per_w // chunk
    mesh = plsc.VectorSubcoreMesh(core_axis_name="c", subcore_axis_name="s")

    @functools.partial(
        pl.kernel,
        mesh=mesh,
        out_type=jax.ShapeDtypeStruct((n_rows, HID), jnp.float32),
        scratch_types=[
            pltpu.VMEM((b_per_w,), jnp.int32),
            pltpu.VMEM((chunk, HID), jnp.float32),
            pltpu.VMEM((chunk, HID), jnp.float32),
            pltpu.SemaphoreType.DMA,
            pltpu.SemaphoreType.DMA,
            pltpu.SemaphoreType.DMA,
            pltpu.SemaphoreType.DMA,
        ],
    )
    def gather_kernel(table_hbm, idx_hbm, out_hbm, idx_v, rows_a, rows_b,
                      sem_ga, sem_gb, sem_oa, sem_ob):
        wid = lax.axis_index("s") * NC + lax.axis_index("c")
        base = wid * b_per_w
        pltpu.sync_copy(idx_hbm.at[pl.ds(base, b_per_w)], idx_v)

        bufs = (rows_a, rows_b)
        gsems = (sem_ga, sem_gb)
        osems = (sem_oa, sem_ob)

        def gather_in(i):
            return pltpu.make_async_copy(
                table_hbm.at[idx_v.at[pl.ds(i * chunk, chunk)]],
                bufs[i % 2], gsems[i % 2])

        def copy_out(i):
            return pltpu.make_async_copy(
                bufs[i % 2], out_hbm.at[pl.ds(base + i * chunk, chunk)],
                osems[i % 2])

        gather_in(0).start()
        for i in range(n_chunks):
            gather_in(i).wait()
            if i >= 1:
                copy_out(i - 1).wait()
            if i + 1 < n_chunks:
                gather_in(i + 1).start()
            copy_out(i).start()
        copy_out(n_chunks - 1).wait()

    return gather_kernel(word_table, ids_flat)


def _tc_fuse_piece(words, position_table, tt_f, token_type_table, ln_gamma,
                   ln_beta, prev, n_rows, piece_rows, seq_len, batch_offset,
                   block_rows, interpret=False):
    n_batch_p = piece_rows // seq_len
    s_blocks = seq_len // block_rows

    def body(w_ref, p_ref, ttf_ref, ttab_ref, g_ref, b_ref, prev_ref, o_ref):
        del prev_ref
        x = w_ref[...] + p_ref[...]
        tt0 = ttab_ref[0, :][None, :]
        dtt = (ttab_ref[1, :] - ttab_ref[0, :])[None, :]
        x = x + tt0 + ttf_ref[...] * dtt
        mu = jnp.mean(x, axis=1, keepdims=True)
        xc = x - mu
        var = jnp.mean(xc * xc, axis=1, keepdims=True)
        y = xc * lax.rsqrt(var + EPS)
        o_ref[...] = y * g_ref[0, :][None, :] + b_ref[0, :][None, :]

    local_block = lambda s, b, _sb=s_blocks: (b * _sb + s, 0)
    out_block = lambda s, b, _sb=s_blocks, _o=batch_offset: ((_o + b) * _sb + s, 0)

    aliases = {6: 0}
    prev_spec = pl.BlockSpec(memory_space=pl.ANY)

    return pl.pallas_call(
        body,
        grid=(s_blocks, n_batch_p),
        in_specs=[
            pl.BlockSpec((block_rows, HID), local_block),
            pl.BlockSpec((block_rows, HID), lambda s, b: (s, 0)),
            pl.BlockSpec((block_rows, 1), local_block),
            pl.BlockSpec((2, HID), lambda s, b: (0, 0)),
            pl.BlockSpec((1, HID), lambda s, b: (0, 0)),
            pl.BlockSpec((1, HID), lambda s, b: (0, 0)),
            prev_spec,
        ],
        out_specs=pl.BlockSpec((block_rows, HID), out_block),
        out_shape=jax.ShapeDtypeStruct((n_rows, HID), jnp.float32),
        input_output_aliases=aliases,
        interpret=interpret,
    )(words, position_table, tt_f, token_type_table, ln_gamma, ln_beta, prev)


def kernel(input_ids, token_type_ids, word_table, position_table,
           token_type_table, ln_gamma, ln_beta):
    B, S = input_ids.shape
    n_rows = B * S
    K = 4
    batch_per_piece = B // K
    piece_rows = n_rows // K

    ids_flat = input_ids.reshape(n_rows).astype(jnp.int32)
    tt_f = token_type_ids.reshape(n_rows, 1).astype(jnp.float32)
    gamma = ln_gamma.reshape(1, HID)
    beta = ln_beta.reshape(1, HID)

    words = [
        _sc_gather(word_table,
                   lax.slice(ids_flat, (k * piece_rows,),
                             ((k + 1) * piece_rows,)),
                   piece_rows, chunk=32)
        for k in range(K)
    ]

    out = jnp.zeros((n_rows, HID), jnp.float32)
    for k in range(K):
        ttf_k = lax.slice(tt_f, (k * piece_rows, 0),
                          ((k + 1) * piece_rows, 1))
        out = _tc_fuse_piece(words[k], position_table, ttf_k,
                             token_type_table, gamma, beta, out, n_rows,
                             piece_rows, S, k * batch_per_piece,
                             block_rows=512)
    return out.reshape(B, S, HID)


# speedup vs baseline: 1.2041x; 1.2041x over previous
import functools
import jax
import jax.numpy as jnp
from jax import lax
from jax.experimental import pallas as pl
from jax.experimental.pallas import tpu as pltpu
from jax.experimental.pallas import tpu_sc as plsc

HID = 1024
EPS = 1e-12

NC = 2
NS = 16
NW = NC * NS


def _sc_gather(word_table, ids_flat, n_rows, chunk):
    b_per_w = n_rows // NW
    n_chunks = b_per_w // chunk
    mesh = plsc.VectorSubcoreMesh(core_axis_name="c", subcore_axis_name="s")

    @functools.partial(
        pl.kernel,
        mesh=mesh,
        out_type=jax.ShapeDtypeStruct((n_rows, HID), jnp.float32),
        scratch_types=[
            pltpu.VMEM((b_per_w,), jnp.int32),
            pltpu.VMEM((chunk, HID), jnp.float32),
            pltpu.VMEM((chunk, HID), jnp.float32),
            pltpu.SemaphoreType.DMA,
            pltpu.SemaphoreType.DMA,
            pltpu.SemaphoreType.DMA,
            pltpu.SemaphoreType.DMA,
        ],
    )
    def gather_kernel(table_hbm, idx_hbm, out_hbm, idx_v, rows_a, rows_b,
                      sem_ga, sem_gb, sem_oa, sem_ob):
        wid = lax.axis_index("s") * NC + lax.axis_index("c")
        base = wid * b_per_w
        pltpu.sync_copy(idx_hbm.at[pl.ds(base, b_per_w)], idx_v)

        bufs = (rows_a, rows_b)
        gsems = (sem_ga, sem_gb)
        osems = (sem_oa, sem_ob)

        def gather_in(i):
            return pltpu.make_async_copy(
                table_hbm.at[idx_v.at[pl.ds(i * chunk, chunk)]],
                bufs[i % 2], gsems[i % 2])

        def copy_out(i):
            return pltpu.make_async_copy(
                bufs[i % 2], out_hbm.at[pl.ds(base + i * chunk, chunk)],
                osems[i % 2])

        gather_in(0).start()
        for i in range(n_chunks):
            gather_in(i).wait()
            if i >= 1:
                copy_out(i - 1).wait()
            if i + 1 < n_chunks:
                gather_in(i + 1).start()
            copy_out(i).start()
        copy_out(n_chunks - 1).wait()

    return gather_kernel(word_table, ids_flat)


def _tc_fuse_piece(words, position_table, tt_f, token_type_table, ln_gamma,
                   ln_beta, prev, n_rows, piece_rows, seq_len, batch_offset,
                   block_rows, interpret=False):
    n_batch_p = piece_rows // seq_len
    s_blocks = seq_len // block_rows

    def body(w_ref, p_ref, ttf_ref, ttab_ref, g_ref, b_ref, prev_ref, o_ref):
        del prev_ref
        x = w_ref[...] + p_ref[...]
        tt0 = ttab_ref[0, :][None, :]
        dtt = (ttab_ref[1, :] - ttab_ref[0, :])[None, :]
        x = x + tt0 + ttf_ref[...] * dtt
        mu = jnp.mean(x, axis=1, keepdims=True)
        xc = x - mu
        var = jnp.mean(xc * xc, axis=1, keepdims=True)
        y = xc * lax.rsqrt(var + EPS)
        o_ref[...] = y * g_ref[0, :][None, :] + b_ref[0, :][None, :]

    local_block = lambda s, b, _sb=s_blocks: (b * _sb + s, 0)
    out_block = lambda s, b, _sb=s_blocks, _o=batch_offset: ((_o + b) * _sb + s, 0)

    if prev is None:
        prev = jnp.zeros((8, HID), jnp.float32)
        aliases = {}
    else:
        aliases = {6: 0}
    prev_spec = pl.BlockSpec(memory_space=pl.ANY)

    return pl.pallas_call(
        body,
        grid=(s_blocks, n_batch_p),
        in_specs=[
            pl.BlockSpec((block_rows, HID), local_block),
            pl.BlockSpec((block_rows, HID), lambda s, b: (s, 0)),
            pl.BlockSpec((block_rows, 1), local_block),
            pl.BlockSpec((2, HID), lambda s, b: (0, 0)),
            pl.BlockSpec((1, HID), lambda s, b: (0, 0)),
            pl.BlockSpec((1, HID), lambda s, b: (0, 0)),
            prev_spec,
        ],
        out_specs=pl.BlockSpec((block_rows, HID), out_block),
        out_shape=jax.ShapeDtypeStruct((n_rows, HID), jnp.float32),
        input_output_aliases=aliases,
        interpret=interpret,
    )(words, position_table, tt_f, token_type_table, ln_gamma, ln_beta, prev)


def kernel(input_ids, token_type_ids, word_table, position_table,
           token_type_table, ln_gamma, ln_beta):
    B, S = input_ids.shape
    n_rows = B * S
    K = 2
    batch_per_piece = B // K
    piece_rows = n_rows // K

    ids_flat = input_ids.reshape(n_rows).astype(jnp.int32)
    tt_f = token_type_ids.reshape(n_rows, 1).astype(jnp.float32)
    gamma = ln_gamma.reshape(1, HID)
    beta = ln_beta.reshape(1, HID)

    words = [
        _sc_gather(word_table,
                   lax.slice(ids_flat, (k * piece_rows,),
                             ((k + 1) * piece_rows,)),
                   piece_rows, chunk=32)
        for k in range(K)
    ]

    out = None
    for k in range(K):
        ttf_k = lax.slice(tt_f, (k * piece_rows, 0),
                          ((k + 1) * piece_rows, 1))
        out = _tc_fuse_piece(words[k], position_table, ttf_k,
                             token_type_table, gamma, beta, out, n_rows,
                             piece_rows, S, k * batch_per_piece,
                             block_rows=512)
    return out.reshape(B, S, HID)


# speedup vs baseline: 1.2284x; 1.0202x over previous
import functools
import jax
import jax.numpy as jnp
from jax import lax
from jax.experimental import pallas as pl
from jax.experimental.pallas import tpu as pltpu
from jax.experimental.pallas import tpu_sc as plsc

HID = 1024
EPS = 1e-12

NC = 2
NS = 16
NW = NC * NS


def _sc_gather(word_table, ids_flat, n_rows, chunk):
    b_per_w = n_rows // NW
    n_chunks = b_per_w // chunk
    mesh = plsc.VectorSubcoreMesh(core_axis_name="c", subcore_axis_name="s")

    @functools.partial(
        pl.kernel,
        mesh=mesh,
        out_type=jax.ShapeDtypeStruct((n_rows, HID), jnp.float32),
        scratch_types=[
            pltpu.VMEM((b_per_w,), jnp.int32),
            pltpu.VMEM((chunk, HID), jnp.float32),
            pltpu.VMEM((chunk, HID), jnp.float32),
            pltpu.SemaphoreType.DMA,
            pltpu.SemaphoreType.DMA,
            pltpu.SemaphoreType.DMA,
            pltpu.SemaphoreType.DMA,
        ],
    )
    def gather_kernel(table_hbm, idx_hbm, out_hbm, idx_v, rows_a, rows_b,
                      sem_ga, sem_gb, sem_oa, sem_ob):
        wid = lax.axis_index("s") * NC + lax.axis_index("c")
        base = wid * b_per_w
        pltpu.sync_copy(idx_hbm.at[pl.ds(base, b_per_w)], idx_v)

        bufs = (rows_a, rows_b)
        gsems = (sem_ga, sem_gb)
        osems = (sem_oa, sem_ob)

        def gather_in(i):
            return pltpu.make_async_copy(
                table_hbm.at[idx_v.at[pl.ds(i * chunk, chunk)]],
                bufs[i % 2], gsems[i % 2])

        def copy_out(i):
            return pltpu.make_async_copy(
                bufs[i % 2], out_hbm.at[pl.ds(base + i * chunk, chunk)],
                osems[i % 2])

        gather_in(0).start()
        for i in range(n_chunks):
            gather_in(i).wait()
            if i >= 1:
                copy_out(i - 1).wait()
            if i + 1 < n_chunks:
                gather_in(i + 1).start()
            copy_out(i).start()
        copy_out(n_chunks - 1).wait()

    return gather_kernel(word_table, ids_flat)


def _tc_fuse_piece(words, position_table, tt_f, token_type_table, ln_gamma,
                   ln_beta, prev, n_rows, piece_rows, seq_len, batch_offset,
                   block_rows, interpret=False):
    n_batch_p = piece_rows // seq_len
    s_blocks = seq_len // block_rows

    def body(w_ref, p_ref, ttf_ref, ttab_ref, g_ref, b_ref, prev_ref, o_ref):
        del prev_ref
        x = w_ref[...] + p_ref[...]
        tt0 = ttab_ref[0, :][None, :]
        dtt = (ttab_ref[1, :] - ttab_ref[0, :])[None, :]
        x = x + tt0 + ttf_ref[...] * dtt
        mu = jnp.mean(x, axis=1, keepdims=True)
        xc = x - mu
        var = jnp.mean(xc * xc, axis=1, keepdims=True)
        y = xc * lax.rsqrt(var + EPS)
        o_ref[...] = y * g_ref[0, :][None, :] + b_ref[0, :][None, :]

    local_block = lambda s, b, _sb=s_blocks: (b * _sb + s, 0)
    out_block = lambda s, b, _sb=s_blocks, _o=batch_offset: ((_o + b) * _sb + s, 0)

    if prev is None:
        prev = jnp.zeros((8, HID), jnp.float32)
        aliases = {}
    else:
        aliases = {6: 0}
    prev_spec = pl.BlockSpec(memory_space=pl.ANY)

    return pl.pallas_call(
        body,
        grid=(s_blocks, n_batch_p),
        in_specs=[
            pl.BlockSpec((block_rows, HID), local_block),
            pl.BlockSpec((block_rows, HID), lambda s, b: (s, 0)),
            pl.BlockSpec((block_rows, 1), local_block),
            pl.BlockSpec((2, HID), lambda s, b: (0, 0)),
            pl.BlockSpec((1, HID), lambda s, b: (0, 0)),
            pl.BlockSpec((1, HID), lambda s, b: (0, 0)),
            prev_spec,
        ],
        out_specs=pl.BlockSpec((block_rows, HID), out_block),
        out_shape=jax.ShapeDtypeStruct((n_rows, HID), jnp.float32),
        input_output_aliases=aliases,
        interpret=interpret,
    )(words, position_table, tt_f, token_type_table, ln_gamma, ln_beta, prev)


def kernel(input_ids, token_type_ids, word_table, position_table,
           token_type_table, ln_gamma, ln_beta):
    B, S = input_ids.shape
    n_rows = B * S
    K = 2
    batch_per_piece = B // K
    piece_rows = n_rows // K

    ids_flat = input_ids.reshape(n_rows).astype(jnp.int32)
    tt_f = token_type_ids.reshape(n_rows, 1).astype(jnp.float32)
    gamma = ln_gamma.reshape(1, HID)
    beta = ln_beta.reshape(1, HID)

    words = [
        _sc_gather(word_table,
                   lax.slice(ids_flat, (k * piece_rows,),
                             ((k + 1) * piece_rows,)),
                   piece_rows, chunk=32)
        for k in range(K)
    ]

    out = None
    for k in range(K):
        ttf_k = lax.slice(tt_f, (k * piece_rows, 0),
                          ((k + 1) * piece_rows, 1))
        out = _tc_fuse_piece(words[k], position_table, ttf_k,
                             token_type_table, gamma, beta, out, n_rows,
                             piece_rows, S, k * batch_per_piece,
                             block_rows=1024)
    return out.reshape(B, S, HID)


# speedup vs baseline: 1.2484x; 1.0162x over previous
import functools
import jax
import jax.numpy as jnp
from jax import lax
from jax.experimental import pallas as pl
from jax.experimental.pallas import tpu as pltpu
from jax.experimental.pallas import tpu_sc as plsc

HID = 1024
EPS = 1e-12

NC = 2
NS = 16
NW = NC * NS


def _sc_gather(word_table, ids_flat, n_rows, chunk):
    b_per_w = n_rows // NW
    n_chunks = b_per_w // chunk
    mesh = plsc.VectorSubcoreMesh(core_axis_name="c", subcore_axis_name="s")

    @functools.partial(
        pl.kernel,
        mesh=mesh,
        out_type=jax.ShapeDtypeStruct((n_rows, HID), jnp.float32),
        scratch_types=[
            pltpu.VMEM((b_per_w,), jnp.int32),
            pltpu.VMEM((chunk, HID), jnp.float32),
            pltpu.VMEM((chunk, HID), jnp.float32),
            pltpu.SemaphoreType.DMA,
            pltpu.SemaphoreType.DMA,
            pltpu.SemaphoreType.DMA,
            pltpu.SemaphoreType.DMA,
        ],
    )
    def gather_kernel(table_hbm, idx_hbm, out_hbm, idx_v, rows_a, rows_b,
                      sem_ga, sem_gb, sem_oa, sem_ob):
        wid = lax.axis_index("s") * NC + lax.axis_index("c")
        base = wid * b_per_w
        pltpu.sync_copy(idx_hbm.at[pl.ds(base, b_per_w)], idx_v)

        bufs = (rows_a, rows_b)
        gsems = (sem_ga, sem_gb)
        osems = (sem_oa, sem_ob)

        def gather_in(i):
            return pltpu.make_async_copy(
                table_hbm.at[idx_v.at[pl.ds(i * chunk, chunk)]],
                bufs[i % 2], gsems[i % 2])

        def copy_out(i):
            return pltpu.make_async_copy(
                bufs[i % 2], out_hbm.at[pl.ds(base + i * chunk, chunk)],
                osems[i % 2])

        gather_in(0).start()
        for i in range(n_chunks):
            gather_in(i).wait()
            if i >= 1:
                copy_out(i - 1).wait()
            if i + 1 < n_chunks:
                gather_in(i + 1).start()
            copy_out(i).start()
        copy_out(n_chunks - 1).wait()

    return gather_kernel(word_table, ids_flat)


def _tc_fuse_piece(words, position_table, tt_f, token_type_table, ln_gamma,
                   ln_beta, prev, n_rows, piece_rows, seq_len, batch_offset,
                   block_rows, interpret=False):
    n_batch_p = piece_rows // seq_len
    s_blocks = seq_len // block_rows

    def body(w_ref, p_ref, ttf_ref, ttab_ref, g_ref, b_ref, prev_ref, o_ref):
        del prev_ref
        x = w_ref[...] + p_ref[...]
        tt0 = ttab_ref[0, :][None, :]
        dtt = (ttab_ref[1, :] - ttab_ref[0, :])[None, :]
        idcol = jnp.transpose(ttf_ref[0], (1, 0))
        x = x + tt0 + idcol * dtt
        mu = jnp.mean(x, axis=1, keepdims=True)
        xc = x - mu
        var = jnp.mean(xc * xc, axis=1, keepdims=True)
        y = xc * lax.rsqrt(var + EPS)
        o_ref[...] = y * g_ref[0, :][None, :] + b_ref[0, :][None, :]

    local_block = lambda s, b, _sb=s_blocks: (b * _sb + s, 0)
    ttf_block = lambda s, b, _sb=s_blocks: (b * _sb + s, 0, 0)
    out_block = lambda s, b, _sb=s_blocks, _o=batch_offset: ((_o + b) * _sb + s, 0)

    if prev is None:
        prev = jnp.zeros((8, HID), jnp.float32)
        aliases = {}
    else:
        aliases = {6: 0}
    prev_spec = pl.BlockSpec(memory_space=pl.ANY)

    return pl.pallas_call(
        body,
        grid=(s_blocks, n_batch_p),
        in_specs=[
            pl.BlockSpec((block_rows, HID), local_block),
            pl.BlockSpec((block_rows, HID), lambda s, b: (s, 0)),
            pl.BlockSpec((1, 1, block_rows), ttf_block),
            pl.BlockSpec((2, HID), lambda s, b: (0, 0)),
            pl.BlockSpec((1, HID), lambda s, b: (0, 0)),
            pl.BlockSpec((1, HID), lambda s, b: (0, 0)),
            prev_spec,
        ],
        out_specs=pl.BlockSpec((block_rows, HID), out_block),
        out_shape=jax.ShapeDtypeStruct((n_rows, HID), jnp.float32),
        input_output_aliases=aliases,
        interpret=interpret,
    )(words, position_table, tt_f, token_type_table, ln_gamma, ln_beta, prev)


def kernel(input_ids, token_type_ids, word_table, position_table,
           token_type_table, ln_gamma, ln_beta):
    B, S = input_ids.shape
    n_rows = B * S
    K = 2
    batch_per_piece = B // K
    piece_rows = n_rows // K

    BR = 1024
    ids_flat = input_ids.reshape(n_rows).astype(jnp.int32)
    tt_f = token_type_ids.astype(jnp.float32).reshape(n_rows // BR, 1, BR)
    gamma = ln_gamma.reshape(1, HID)
    beta = ln_beta.reshape(1, HID)

    words = [
        _sc_gather(word_table,
                   lax.slice(ids_flat, (k * piece_rows,),
                             ((k + 1) * piece_rows,)),
                   piece_rows, chunk=32)
        for k in range(K)
    ]

    piece_blocks = piece_rows // BR
    out = None
    for k in range(K):
        ttf_k = lax.slice(tt_f, (k * piece_blocks, 0, 0),
                          ((k + 1) * piece_blocks, 1, BR))
        out = _tc_fuse_piece(words[k], position_table, ttf_k,
                             token_type_table, gamma, beta, out, n_rows,
                             piece_rows, S, k * batch_per_piece,
                             block_rows=BR)
    return out.reshape(B, S, HID)
